# baseline (speedup 1.0000x reference)
import jax
import jax.numpy as jnp
from jax import lax
from jax.experimental import pallas as pl
from jax.experimental.pallas import tpu as pltpu
from jax.experimental.pallas import tpu_sc as plsc

B, F, V, D = 4096, 26, 1000000, 13
L = 16
NC, NS = 2, 16
NW = NC * NS
BPW = B // NW
NCH = BPW // L


def _body(xs_hbm, xd_hbm, tab_hbm, w_hbm, out_hbm,
          xs_v, xd_v, w_v, idx_v, gath_v, out_v, sem):
    wid = lax.axis_index("s") * NC + lax.axis_index("c")
    base = wid * BPW

    pltpu.sync_copy(xs_hbm.at[pl.ds(base * F, BPW * F)], xs_v)
    pltpu.sync_copy(xd_hbm.at[pl.ds(base * D, BPW * D)], xd_v)
    pltpu.sync_copy(w_hbm, w_v)

    iota = lax.broadcasted_iota(jnp.int32, (L,), 0)

    def build_f(f, _):
        off = f * V

        def build_c(c, _):
            rows = c * L + iota
            v = plsc.load_gather(xs_v, [rows * F + f])
            idx_v[pl.ds(f * BPW + c * L, L)] = v + off
            return 0

        return lax.fori_loop(0, NCH, build_c, 0)

    lax.fori_loop(0, F, build_f, 0)

    pltpu.async_copy(tab_hbm.at[idx_v], gath_v, sem).wait()

    def red_c(c, _):
        rows = c * L + iota

        def red_f(f, s):
            return s + gath_v[pl.ds(f * BPW + c * L, L)]

        s = lax.fori_loop(0, F, red_f, jnp.zeros((L,), jnp.float32))

        def red_d(d, s):
            dcol = jnp.zeros((L,), jnp.int32) + d
            xv = plsc.load_gather(xd_v, [rows * D + d])
            wv = plsc.load_gather(w_v, [dcol])
            return s + xv * wv

        s = lax.fori_loop(0, D, red_d, s)
        out_v[pl.ds(c * L, L)] = s
        return 0

    lax.fori_loop(0, NCH, red_c, 0)

    pltpu.sync_copy(out_v, out_hbm.at[pl.ds(base, BPW)])


def kernel(x_sparse, x_dense, emb_table, dense_weight):
    tab_flat = emb_table.reshape(-1)
    xs_flat = x_sparse.reshape(-1)
    xd_flat = x_dense.reshape(-1)
    w_pad = jnp.pad(dense_weight.reshape(-1), (0, L - D))
    mesh = plsc.VectorSubcoreMesh(core_axis_name="c", subcore_axis_name="s")
    run = pl.kernel(
        _body,
        out_type=jax.ShapeDtypeStruct((B,), jnp.float32),
        mesh=mesh,
        compiler_params=pltpu.CompilerParams(needs_layout_passes=False),
        scratch_types=[
            pltpu.VMEM((BPW * F,), jnp.int32),
            pltpu.VMEM((BPW * D,), jnp.float32),
            pltpu.VMEM((L,), jnp.float32),
            pltpu.VMEM((F * BPW,), jnp.int32),
            pltpu.VMEM((F * BPW,), jnp.float32),
            pltpu.VMEM((BPW,), jnp.float32),
            pltpu.SemaphoreType.DMA,
        ],
    )
    out = run(xs_flat, xd_flat, tab_flat, w_pad)
    return out.reshape(B, 1)


# speedup vs baseline: 5.1394x; 5.1394x over previous
import jax
import jax.numpy as jnp
from jax import lax
from jax.experimental import pallas as pl
from jax.experimental.pallas import tpu as pltpu
from jax.experimental.pallas import tpu_sc as plsc

B, F, V, D = 4096, 26, 1000000, 13
L = 16
NC, NS = 2, 16
NW = NC * NS
BPW = B // NW
NCH = BPW // L


def _sc_body(*refs):
    xs_hbm, xd_hbm = refs[0], refs[1]
    tabs = refs[2:2 + F]
    w_hbm = refs[2 + F]
    out_hbm = refs[3 + F]
    xs_v, xd_v, w_v, idx_v, gath_v, out_v, sem = refs[4 + F:]

    wid = lax.axis_index("s") * NC + lax.axis_index("c")
    base = wid * BPW

    pltpu.sync_copy(xs_hbm.at[pl.ds(base * F, BPW * F)], xs_v)
    pltpu.sync_copy(xd_hbm.at[pl.ds(base * D, BPW * D)], xd_v)
    pltpu.sync_copy(w_hbm, w_v)

    iota = lax.broadcasted_iota(jnp.int32, (L,), 0)

    def build_f(f, _):
        def build_c(c, _):
            rows = c * L + iota
            idx_v[f, pl.ds(c * L, L)] = plsc.load_gather(xs_v, [rows * F + f])
            return 0

        return lax.fori_loop(0, NCH, build_c, 0)

    lax.fori_loop(0, F, build_f, 0)

    copies = [
        pltpu.async_copy(tabs[f].at[idx_v.at[f]], gath_v.at[f], sem)
        for f in range(F)
    ]
    for c in copies:
        c.wait()

    def red_c(c, _):
        rows = c * L + iota

        def red_f(f, s):
            return s + gath_v[f, pl.ds(c * L, L)]

        s = lax.fori_loop(0, F, red_f, jnp.zeros((L,), jnp.float32))

        def red_d(d, s):
            dcol = jnp.zeros((L,), jnp.int32) + d
            xv = plsc.load_gather(xd_v, [rows * D + d])
            wv = plsc.load_gather(w_v, [dcol])
            return s + xv * wv

        s = lax.fori_loop(0, D, red_d, s)
        out_v[pl.ds(c * L, L)] = s
        return 0

    lax.fori_loop(0, NCH, red_c, 0)

    pltpu.sync_copy(out_v, out_hbm.at[pl.ds(base, BPW)])


def kernel(x_sparse, x_dense, emb_table, dense_weight):
    tabs = [emb_table[f].reshape(-1) for f in range(F)]
    xs_flat = x_sparse.reshape(-1)
    xd_flat = x_dense.reshape(-1)
    w_pad = jnp.pad(dense_weight.reshape(-1), (0, L - D))
    mesh = plsc.VectorSubcoreMesh(core_axis_name="c", subcore_axis_name="s")
    run = pl.kernel(
        _sc_body,
        out_type=jax.ShapeDtypeStruct((B,), jnp.float32),
        mesh=mesh,
        compiler_params=pltpu.CompilerParams(needs_layout_passes=False),
        scratch_types=[
            pltpu.VMEM((BPW * F,), jnp.int32),
            pltpu.VMEM((BPW * D,), jnp.float32),
            pltpu.VMEM((L,), jnp.float32),
            pltpu.VMEM((F, BPW), jnp.int32),
            pltpu.VMEM((F, BPW), jnp.float32),
            pltpu.VMEM((BPW,), jnp.float32),
            pltpu.SemaphoreType.DMA,
        ],
    )
    out = run(xs_flat, xd_flat, *tabs, w_pad)
    return out.reshape(B, 1)
